# Initial kernel scaffold; baseline (speedup 1.0000x reference)
import jax
import jax.numpy as jnp
from jax.experimental import pallas as pl


def kernel(x, adj, poly_item, W1, b1, W2, b2, filter_param):
    raise NotImplementedError("write your pallas kernel here")


# speedup vs baseline: 2.1575x; 2.1575x over previous
import math

import jax
import jax.numpy as jnp
from jax.experimental import pallas as pl
from jax.experimental.pallas import tpu as pltpu

_K = 5
_N = 4096
_D_IN = 512
_D_HID = 256
_D_OUT = 64
_BR = 512
_RB = _N // _BR
_STEPS = _K + 1


def _body(fp_ref, comb_ref, x_ref, w1_ref, b1_ref, w2_ref, b2_ref,
          a_ref, p_ref, out_ref, v0, t0, v1, t1):
    s = pl.program_id(0)
    r = pl.program_id(1)
    row = r * _BR

    @pl.when(s == 0)
    def _mlp():
        h1 = jnp.dot(x_ref[...], w1_ref[...], preferred_element_type=jnp.float32)
        h1 = jnp.maximum(h1 + b1_ref[...], 0.0)
        h = jnp.dot(h1, w2_ref[...], preferred_element_type=jnp.float32) + b2_ref[...]
        cK = jnp.maximum(fp_ref[_K, 0], 0.0) * comb_ref[_K, 0]
        v0[pl.ds(row, _BR), :] = h
        t0[pl.ds(row, _BR), :] = cK * h

    def _step(sv, st, dv, dt):
        i = _K - s
        coef = jnp.maximum(fp_ref[i, 0], 0.0) * comb_ref[i, 0]
        v_new = jnp.dot(a_ref[...], sv[...], preferred_element_type=jnp.float32)
        t_new = jnp.dot(p_ref[...], st[...], preferred_element_type=jnp.float32)
        t_new = t_new + coef * v_new
        dv[pl.ds(row, _BR), :] = v_new
        dt[pl.ds(row, _BR), :] = t_new

        @pl.when(s == _K)
        def _out():
            m = jnp.max(t_new, axis=1, keepdims=True)
            lse = jnp.log(jnp.sum(jnp.exp(t_new - m), axis=1, keepdims=True)) + m
            out_ref[...] = t_new - lse

    @pl.when((s > 0) & (s % 2 == 1))
    def _odd():
        _step(v0, t0, v1, t1)

    @pl.when((s > 0) & (s % 2 == 0))
    def _even():
        _step(v1, t1, v0, t0)


def kernel(x, adj, poly_item, W1, b1, W2, b2, filter_param):
    comb = jnp.asarray(
        [[math.comb(_K, i) / (2.0 ** _K)] for i in range(_K + 1)], dtype=jnp.float32)
    b1r = b1.reshape(1, _D_HID)
    b2r = b2.reshape(1, _D_OUT)

    grid = (_STEPS, _RB)
    out = pl.pallas_call(
        _body,
        grid=grid,
        in_specs=[
            pl.BlockSpec(memory_space=pltpu.SMEM),
            pl.BlockSpec(memory_space=pltpu.SMEM),
            pl.BlockSpec((_BR, _D_IN), lambda s, r: (jnp.where(s == 0, r, _RB - 1), 0)),
            pl.BlockSpec((_D_IN, _D_HID), lambda s, r: (0, 0)),
            pl.BlockSpec((1, _D_HID), lambda s, r: (0, 0)),
            pl.BlockSpec((_D_HID, _D_OUT), lambda s, r: (0, 0)),
            pl.BlockSpec((1, _D_OUT), lambda s, r: (0, 0)),
            pl.BlockSpec((_BR, _N), lambda s, r: (jnp.where(s == 0, 0, r), 0)),
            pl.BlockSpec((_BR, _N), lambda s, r: (jnp.where(s == 0, 0, r), 0)),
        ],
        out_specs=pl.BlockSpec((_BR, _D_OUT), lambda s, r: (jnp.where(s == _K, r, 0), 0)),
        out_shape=jax.ShapeDtypeStruct((_N, _D_OUT), jnp.float32),
        scratch_shapes=[
            pltpu.VMEM((_N, _D_OUT), jnp.float32),
            pltpu.VMEM((_N, _D_OUT), jnp.float32),
            pltpu.VMEM((_N, _D_OUT), jnp.float32),
            pltpu.VMEM((_N, _D_OUT), jnp.float32),
        ],
        compiler_params=pltpu.CompilerParams(
            dimension_semantics=("arbitrary", "arbitrary"),
        ),
    )(filter_param, comb, x, W1, b1r, W2, b2r, adj, poly_item)
    return out


# speedup vs baseline: 2.4502x; 1.1357x over previous
import math

import jax
import jax.numpy as jnp
from jax.experimental import pallas as pl
from jax.experimental.pallas import tpu as pltpu

_K = 5
_N = 4096
_D_IN = 512
_D_HID = 256
_D_OUT = 64
_BR = 256
_RB = _N // _BR
_STEPS = _K + 1


def _body(fp_ref, comb_ref, x_ref, w1_ref, b1_ref, w2_ref, b2_ref,
          a_ref, p_ref, out_ref, a_scr, v0, t0, v1, t1, vb, tb):
    s = pl.program_id(0)
    r = pl.program_id(1)
    row = r * _BR

    @pl.when(s == 0)
    def _prologue():
        a_scr[pl.ds(row, _BR), :] = a_ref[...].astype(jnp.bfloat16)
        h1 = jnp.dot(x_ref[...], w1_ref[...], preferred_element_type=jnp.float32)
        h1 = jnp.maximum(h1 + b1_ref[...], 0.0)
        h = jnp.dot(h1, w2_ref[...], preferred_element_type=jnp.float32) + b2_ref[...]
        cK = jnp.maximum(fp_ref[_K, 0], 0.0) * comb_ref[_K, 0]
        v0[pl.ds(row, _BR), :] = h
        t0[pl.ds(row, _BR), :] = cK * h

    def _step(sv, st, dv, dt):
        @pl.when(r == 0)
        def _cast():
            sv_f = sv[...]
            vh = sv_f.astype(jnp.bfloat16)
            vl = (sv_f - vh.astype(jnp.float32)).astype(jnp.bfloat16)
            vb[...] = jnp.concatenate([vh, vl], axis=1)
            st_f = st[...]
            th = st_f.astype(jnp.bfloat16)
            tl = (st_f - th.astype(jnp.float32)).astype(jnp.bfloat16)
            tb[...] = jnp.concatenate([th, tl], axis=1)

        i = _K - s
        coef = jnp.maximum(fp_ref[i, 0], 0.0) * comb_ref[i, 0]
        v_pair = jnp.dot(a_scr[pl.ds(row, _BR), :], vb[...],
                         preferred_element_type=jnp.float32)
        v_new = v_pair[:, :_D_OUT] + v_pair[:, _D_OUT:]
        t_pair = jnp.dot(p_ref[...], tb[...], preferred_element_type=jnp.float32)
        t_new = t_pair[:, :_D_OUT] + t_pair[:, _D_OUT:] + coef * v_new
        dv[pl.ds(row, _BR), :] = v_new
        dt[pl.ds(row, _BR), :] = t_new

        @pl.when(s == _K)
        def _out():
            m = jnp.max(t_new, axis=1, keepdims=True)
            lse = jnp.log(jnp.sum(jnp.exp(t_new - m), axis=1, keepdims=True)) + m
            out_ref[...] = t_new - lse

    @pl.when((s > 0) & (s % 2 == 1))
    def _odd():
        _step(v0, t0, v1, t1)

    @pl.when((s > 0) & (s % 2 == 0))
    def _even():
        _step(v1, t1, v0, t0)


def kernel(x, adj, poly_item, W1, b1, W2, b2, filter_param):
    comb = jnp.asarray(
        [[math.comb(_K, i) / (2.0 ** _K)] for i in range(_K + 1)], dtype=jnp.float32)
    b1r = b1.reshape(1, _D_HID)
    b2r = b2.reshape(1, _D_OUT)
    p_bf = poly_item.astype(jnp.bfloat16)

    grid = (_STEPS, _RB)
    out = pl.pallas_call(
        _body,
        grid=grid,
        in_specs=[
            pl.BlockSpec(memory_space=pltpu.SMEM),
            pl.BlockSpec(memory_space=pltpu.SMEM),
            pl.BlockSpec((_BR, _D_IN), lambda s, r: (jnp.where(s == 0, r, _RB - 1), 0)),
            pl.BlockSpec((_D_IN, _D_HID), lambda s, r: (0, 0)),
            pl.BlockSpec((1, _D_HID), lambda s, r: (0, 0)),
            pl.BlockSpec((_D_HID, _D_OUT), lambda s, r: (0, 0)),
            pl.BlockSpec((1, _D_OUT), lambda s, r: (0, 0)),
            pl.BlockSpec((_BR, _N), lambda s, r: (jnp.where(s == 0, r, _RB - 1), 0)),
            pl.BlockSpec((_BR, _N), lambda s, r: (jnp.where(s == 0, 0, r), 0)),
        ],
        out_specs=pl.BlockSpec((_BR, _D_OUT), lambda s, r: (jnp.where(s == _K, r, 0), 0)),
        out_shape=jax.ShapeDtypeStruct((_N, _D_OUT), jnp.float32),
        scratch_shapes=[
            pltpu.VMEM((_N, _N), jnp.bfloat16),
            pltpu.VMEM((_N, _D_OUT), jnp.float32),
            pltpu.VMEM((_N, _D_OUT), jnp.float32),
            pltpu.VMEM((_N, _D_OUT), jnp.float32),
            pltpu.VMEM((_N, _D_OUT), jnp.float32),
            pltpu.VMEM((_N, 2 * _D_OUT), jnp.bfloat16),
            pltpu.VMEM((_N, 2 * _D_OUT), jnp.bfloat16),
        ],
        compiler_params=pltpu.CompilerParams(
            dimension_semantics=("arbitrary", "arbitrary"),
        ),
    )(filter_param, comb, x, W1, b1r, W2, b2r, adj, p_bf)
    return out
